# Initial kernel scaffold; baseline (speedup 1.0000x reference)
import jax
import jax.numpy as jnp
from jax.experimental import pallas as pl


def kernel(obs, edge_index, W1, b1, W2, b2):
    raise NotImplementedError("write your pallas kernel here")


# speedup vs baseline: 41.3932x; 41.3932x over previous
import functools

import jax
import jax.numpy as jnp
from jax import lax
from jax.experimental import pallas as pl
from jax.experimental.pallas import tpu as pltpu
from jax.experimental.pallas import tpu_sc as plsc

N = 10000
E = 320000
NP = 10240
D1 = 128
D2 = 64
NC = 2
NS = 16
NW = NC * NS
EW = E // NW
CH = 125
NCHUNK = EW // CH
E16 = E // NS
SL = NP // NS

_mesh = plsc.VectorSubcoreMesh(core_axis_name="c", subcore_axis_name="s")
_sc_params = pltpu.CompilerParams(needs_layout_passes=False,
                                  use_tc_tiling_on_sc=False)


@functools.partial(
    pl.kernel,
    out_type=jax.ShapeDtypeStruct((NW, NP), jnp.float32),
    mesh=_mesh,
    compiler_params=_sc_params,
    scratch_types=[
        pltpu.VMEM((EW,), jnp.int32),
        pltpu.VMEM((NP,), jnp.float32),
    ],
)
def _deg_kernel(dst_hbm, out_hbm, dstv, acc):
    c = lax.axis_index("c")
    s = lax.axis_index("s")
    w = s * NC + c
    pltpu.sync_copy(dst_hbm.at[pl.ds(w * EW, EW)], dstv)

    def zero(i, carry):
        acc[pl.ds(i * 16, 16)] = jnp.zeros((16,), jnp.float32)
        return carry

    lax.fori_loop(0, NP // 16, zero, 0)
    ones = jnp.full((16,), 1.0, jnp.float32)

    def body(i, carry):
        d16 = dstv[pl.ds(i * 16, 16)]
        plsc.addupdate_scatter(acc, [d16], ones)
        return carry

    lax.fori_loop(0, EW // 16, body, 0)
    pltpu.sync_copy(acc, out_hbm.at[w])


def _mm1_body(parts_ref, obs_ref, w1_ref, h_ref, dis_ref):
    deg = jnp.sum(parts_ref[...], axis=1, keepdims=True) + 1.0
    dis = lax.rsqrt(deg)
    u = jnp.dot(obs_ref[...], w1_ref[...], preferred_element_type=jnp.float32)
    h_ref[...] = u * dis
    dis_ref[...] = dis


_mm1 = pl.pallas_call(
    _mm1_body,
    out_shape=(
        jax.ShapeDtypeStruct((N, D2), jnp.float32),
        jax.ShapeDtypeStruct((N, 1), jnp.float32),
    ),
)


@functools.partial(
    pl.kernel,
    out_type=jax.ShapeDtypeStruct((NC, NP, D2), jnp.float32),
    mesh=_mesh,
    compiler_params=_sc_params,
    scratch_types=[
        pltpu.VMEM((NCHUNK, CH), jnp.int32),
        pltpu.VMEM((NCHUNK, CH), jnp.int32),
        pltpu.VMEM((CH, D2), jnp.float32),
        pltpu.VMEM_SHARED((NP, D2), jnp.float32),
        pltpu.SemaphoreType.DMA,
    ],
)
def _agg_kernel(h_hbm, src_hbm, dst_hbm, zeros_hbm, out_hbm,
                srcv, dstv, rows, accsh, sem):
    c = lax.axis_index("c")
    s = lax.axis_index("s")
    w = s * NC + c
    myslice = pl.ds(s * SL, SL)
    pltpu.sync_copy(zeros_hbm, accsh.at[myslice])
    pltpu.sync_copy(src_hbm.at[pl.ds(w * NCHUNK, NCHUNK), :], srcv)
    pltpu.sync_copy(dst_hbm.at[pl.ds(w * NCHUNK, NCHUNK), :], dstv)
    plsc.subcore_barrier()

    def chunk(i, carry):
        pltpu.async_copy(h_hbm.at[srcv.at[i]], rows, sem).wait()
        pltpu.sync_copy(rows, accsh.at[dstv.at[i]], add=True)
        return carry

    lax.fori_loop(0, NCHUNK, chunk, 0)
    plsc.subcore_barrier()
    pltpu.sync_copy(accsh.at[myslice], out_hbm.at[c, myslice, :])


def _mm2_body(a0_ref, a1_ref, h_ref, dis_ref, b1_ref, w2_ref, zp_ref):
    dis = dis_ref[...]
    h1 = jnp.maximum(dis * (a0_ref[...] + a1_ref[...] + h_ref[...])
                     + b1_ref[...], 0.0)
    z = jnp.dot(h1, w2_ref[...], preferred_element_type=jnp.float32)
    zp_ref[...] = z * dis


_mm2 = pl.pallas_call(
    _mm2_body,
    out_shape=jax.ShapeDtypeStruct((N, 1), jnp.float32),
)


@functools.partial(
    pl.kernel,
    out_type=jax.ShapeDtypeStruct((NP,), jnp.float32),
    mesh=_mesh,
    compiler_params=_sc_params,
    scratch_types=[
        pltpu.VMEM((E16,), jnp.int32),
        pltpu.VMEM((E16,), jnp.int32),
        pltpu.VMEM((NP,), jnp.float32),
        pltpu.VMEM((NP,), jnp.float32),
        pltpu.VMEM((NP,), jnp.float32),
        pltpu.VMEM((NP,), jnp.float32),
        pltpu.VMEM((SL,), jnp.float32),
        pltpu.VMEM((16,), jnp.float32),
        pltpu.VMEM_SHARED((NS * NP,), jnp.float32),
    ],
)
def _agg2_kernel(src_hbm, dst_hbm, zp_hbm, dis_hbm, b2_hbm, out_hbm,
                 srcv, dstv, zv, acc, disv, rbuf, obuf, b2v, accsh):
    c = lax.axis_index("c")
    s = lax.axis_index("s")

    @pl.when(c == 0)
    def _scatter():
        pltpu.sync_copy(src_hbm.at[pl.ds(s * E16, E16)], srcv)
        pltpu.sync_copy(dst_hbm.at[pl.ds(s * E16, E16)], dstv)
        pltpu.sync_copy(zp_hbm, zv)

        def zero(i, carry):
            acc[pl.ds(i * 16, 16)] = jnp.zeros((16,), jnp.float32)
            return carry

        lax.fori_loop(0, NP // 16, zero, 0)

        def body(i, carry):
            s16 = srcv[pl.ds(i * 16, 16)]
            d16 = dstv[pl.ds(i * 16, 16)]
            v = plsc.load_gather(zv, [s16])
            plsc.addupdate_scatter(acc, [d16], v)
            return carry

        lax.fori_loop(0, E16 // 16, body, 0)
        pltpu.sync_copy(acc, accsh.at[pl.ds(s * NP, NP)])

    plsc.subcore_barrier()

    @pl.when(c == 0)
    def _reduce():
        pltpu.sync_copy(dis_hbm, disv)
        pltpu.sync_copy(b2_hbm, b2v)
        for r in range(NS):
            pltpu.sync_copy(accsh.at[pl.ds(r * NP + s * SL, SL)],
                            rbuf.at[pl.ds(r * SL, SL)])
        b2 = b2v[pl.ds(0, 16)]

        def fin(j, carry):
            tot = rbuf[pl.ds(j * 16, 16)]
            for r in range(1, NS):
                tot = tot + rbuf[pl.ds(r * SL + j * 16, 16)]
            node = s * SL + j * 16
            d16 = disv[pl.ds(node, 16)]
            z16 = zv[pl.ds(node, 16)]
            obuf[pl.ds(j * 16, 16)] = jnp.maximum(d16 * (tot + z16) + b2, 0.0)
            return carry

        lax.fori_loop(0, SL // 16, fin, 0)
        pltpu.sync_copy(obuf, out_hbm.at[pl.ds(s * SL, SL)])


def kernel(obs, edge_index, W1, b1, W2, b2):
    src = edge_index[0].astype(jnp.int32)
    dst = edge_index[1].astype(jnp.int32)

    deg_parts = _deg_kernel(dst)
    degT = deg_parts[:, :N].T

    hprime, dis = _mm1(degT, obs, W1)

    src2 = src.reshape(E // CH, CH)
    dst2 = dst.reshape(E // CH, CH)
    zeros_init = jnp.zeros((SL, D2), jnp.float32)
    parts = _agg_kernel(hprime, src2, dst2, zeros_init)

    zp = _mm2(parts[0, :N], parts[1, :N], hprime, dis,
              b1.reshape(1, D2), W2[:, -1:])

    zpad = jnp.pad(zp[:, 0], (0, NP - N))
    dpad = jnp.pad(dis[:, 0], (0, NP - N))
    b2v = jnp.full((16,), b2[-1], jnp.float32)
    out_pad = _agg2_kernel(src, dst, zpad, dpad, b2v)
    return out_pad[:N]


# speedup vs baseline: 53.7809x; 1.2993x over previous
import functools

import jax
import jax.numpy as jnp
from jax import lax
from jax.experimental import pallas as pl
from jax.experimental.pallas import tpu as pltpu
from jax.experimental.pallas import tpu_sc as plsc

N = 10000
E = 320000
NP = 10240
D1 = 128
D2 = 64
NC = 2
NS = 16
NW = NC * NS
EW = E // NW
CH = 125
NCHUNK = EW // CH
E16 = E // NS
SL = NP // NS

_mesh = plsc.VectorSubcoreMesh(core_axis_name="c", subcore_axis_name="s")
_sc_params = pltpu.CompilerParams(needs_layout_passes=False,
                                  use_tc_tiling_on_sc=False)


@functools.partial(
    pl.kernel,
    out_type=jax.ShapeDtypeStruct((NW, NP), jnp.float32),
    mesh=_mesh,
    compiler_params=_sc_params,
    scratch_types=[
        pltpu.VMEM((EW,), jnp.int32),
        pltpu.VMEM((NP,), jnp.float32),
    ],
)
def _deg_kernel(dst_hbm, out_hbm, dstv, acc):
    c = lax.axis_index("c")
    s = lax.axis_index("s")
    w = s * NC + c
    pltpu.sync_copy(dst_hbm.at[pl.ds(w * EW, EW)], dstv)

    def zero(i, carry):
        acc[pl.ds(i * 16, 16)] = jnp.zeros((16,), jnp.float32)
        return carry

    lax.fori_loop(0, NP // 16, zero, 0)
    ones = jnp.full((16,), 1.0, jnp.float32)

    def body(i, carry):
        d16 = dstv[pl.ds(i * 16, 16)]
        plsc.addupdate_scatter(acc, [d16], ones)
        return carry

    lax.fori_loop(0, EW // 16, body, 0)
    pltpu.sync_copy(acc, out_hbm.at[w])


def _mm1_body(parts_ref, obs_ref, w1_ref, h_ref, dis_ref):
    deg = jnp.sum(parts_ref[...], axis=1, keepdims=True) + 1.0
    dis = lax.rsqrt(deg)
    u = jnp.dot(obs_ref[...], w1_ref[...], preferred_element_type=jnp.float32)
    h_ref[...] = u * dis
    dis_ref[...] = dis


_mm1 = pl.pallas_call(
    _mm1_body,
    out_shape=(
        jax.ShapeDtypeStruct((N, D2), jnp.float32),
        jax.ShapeDtypeStruct((N, 1), jnp.float32),
    ),
)


@functools.partial(
    pl.kernel,
    out_type=jax.ShapeDtypeStruct((NC, NP, D2), jnp.float32),
    mesh=_mesh,
    compiler_params=_sc_params,
    scratch_types=[
        pltpu.VMEM((NCHUNK, CH), jnp.int32),
        pltpu.VMEM((NCHUNK, CH), jnp.int32),
        pltpu.VMEM((CH, D2), jnp.float32),
        pltpu.VMEM((CH, D2), jnp.float32),
        pltpu.VMEM_SHARED((NP, D2), jnp.float32),
        pltpu.SemaphoreType.DMA,
        pltpu.SemaphoreType.DMA,
    ],
)
def _agg_kernel(h_hbm, src_hbm, dst_hbm, zeros_hbm, out_hbm,
                srcv, dstv, rows0, rows1, accsh, sem0, sem1):
    c = lax.axis_index("c")
    s = lax.axis_index("s")
    w = s * NC + c
    myslice = pl.ds(s * SL, SL)
    pltpu.sync_copy(zeros_hbm, accsh.at[myslice])
    pltpu.sync_copy(src_hbm.at[pl.ds(w * NCHUNK, NCHUNK), :], srcv)
    pltpu.sync_copy(dst_hbm.at[pl.ds(w * NCHUNK, NCHUNK), :], dstv)
    plsc.subcore_barrier()

    pltpu.async_copy(h_hbm.at[srcv.at[0]], rows0, sem0)

    def pair(i, carry):
        i2 = i * 2
        pltpu.async_copy(h_hbm.at[srcv.at[i2 + 1]], rows1, sem1)
        pltpu.make_async_copy(h_hbm.at[srcv.at[i2]], rows0, sem0).wait()
        pltpu.sync_copy(rows0, accsh.at[dstv.at[i2]], add=True)

        @pl.when(i2 + 2 < NCHUNK)
        def _():
            pltpu.async_copy(h_hbm.at[srcv.at[i2 + 2]], rows0, sem0)

        pltpu.make_async_copy(h_hbm.at[srcv.at[i2 + 1]], rows1, sem1).wait()
        pltpu.sync_copy(rows1, accsh.at[dstv.at[i2 + 1]], add=True)
        return carry

    lax.fori_loop(0, NCHUNK // 2, pair, 0)
    plsc.subcore_barrier()
    pltpu.sync_copy(accsh.at[myslice], out_hbm.at[c, myslice, :])


def _mm2_body(parts_ref, h_ref, dis_ref, b1_ref, w2_ref, zp_ref):
    dis = dis_ref[...]
    h1 = jnp.maximum(dis * (parts_ref[0, :N, :] + parts_ref[1, :N, :]
                            + h_ref[...]) + b1_ref[...], 0.0)
    z = jnp.dot(h1, w2_ref[...], preferred_element_type=jnp.float32)
    zp_ref[...] = z * dis


_mm2 = pl.pallas_call(
    _mm2_body,
    out_shape=jax.ShapeDtypeStruct((N, 1), jnp.float32),
)


@functools.partial(
    pl.kernel,
    out_type=jax.ShapeDtypeStruct((NP,), jnp.float32),
    mesh=_mesh,
    compiler_params=_sc_params,
    scratch_types=[
        pltpu.VMEM((E16,), jnp.int32),
        pltpu.VMEM((E16,), jnp.int32),
        pltpu.VMEM((NP,), jnp.float32),
        pltpu.VMEM((NP,), jnp.float32),
        pltpu.VMEM((NP,), jnp.float32),
        pltpu.VMEM((NP,), jnp.float32),
        pltpu.VMEM((SL,), jnp.float32),
        pltpu.VMEM((16,), jnp.float32),
        pltpu.VMEM_SHARED((NS * NP,), jnp.float32),
    ],
)
def _agg2_kernel(src_hbm, dst_hbm, zp_hbm, dis_hbm, b2_hbm, out_hbm,
                 srcv, dstv, zv, acc, disv, rbuf, obuf, b2v, accsh):
    c = lax.axis_index("c")
    s = lax.axis_index("s")

    @pl.when(c == 0)
    def _scatter():
        pltpu.sync_copy(src_hbm.at[pl.ds(s * E16, E16)], srcv)
        pltpu.sync_copy(dst_hbm.at[pl.ds(s * E16, E16)], dstv)
        pltpu.sync_copy(zp_hbm, zv)

        def zero(i, carry):
            acc[pl.ds(i * 16, 16)] = jnp.zeros((16,), jnp.float32)
            return carry

        lax.fori_loop(0, NP // 16, zero, 0)

        def body(i, carry):
            s16 = srcv[pl.ds(i * 16, 16)]
            d16 = dstv[pl.ds(i * 16, 16)]
            v = plsc.load_gather(zv, [s16])
            plsc.addupdate_scatter(acc, [d16], v)
            return carry

        lax.fori_loop(0, E16 // 16, body, 0)
        pltpu.sync_copy(acc, accsh.at[pl.ds(s * NP, NP)])

    plsc.subcore_barrier()

    @pl.when(c == 0)
    def _reduce():
        pltpu.sync_copy(dis_hbm, disv)
        pltpu.sync_copy(b2_hbm, b2v)
        for r in range(NS):
            pltpu.sync_copy(accsh.at[pl.ds(r * NP + s * SL, SL)],
                            rbuf.at[pl.ds(r * SL, SL)])
        b2 = b2v[pl.ds(0, 16)]

        def fin(j, carry):
            tot = rbuf[pl.ds(j * 16, 16)]
            for r in range(1, NS):
                tot = tot + rbuf[pl.ds(r * SL + j * 16, 16)]
            node = s * SL + j * 16
            d16 = disv[pl.ds(node, 16)]
            z16 = zv[pl.ds(node, 16)]
            obuf[pl.ds(j * 16, 16)] = jnp.maximum(d16 * (tot + z16) + b2, 0.0)
            return carry

        lax.fori_loop(0, SL // 16, fin, 0)
        pltpu.sync_copy(obuf, out_hbm.at[pl.ds(s * SL, SL)])


def kernel(obs, edge_index, W1, b1, W2, b2):
    src = edge_index[0].astype(jnp.int32)
    dst = edge_index[1].astype(jnp.int32)

    deg_parts = _deg_kernel(dst)
    degT = deg_parts[:, :N].T

    hprime, dis = _mm1(degT, obs, W1)

    src2 = src.reshape(E // CH, CH)
    dst2 = dst.reshape(E // CH, CH)
    zeros_init = jnp.zeros((SL, D2), jnp.float32)
    parts = _agg_kernel(hprime, src2, dst2, zeros_init)

    zp = _mm2(parts, hprime, dis,
              b1.reshape(1, D2), W2[:, -1:])

    zpad = jnp.pad(zp[:, 0], (0, NP - N))
    dpad = jnp.pad(dis[:, 0], (0, NP - N))
    b2v = jnp.full((16,), b2[-1], jnp.float32)
    out_pad = _agg2_kernel(src, dst, zpad, dpad, b2v)
    return out_pad[:N]
